# baseline (speedup 1.0000x reference)
import functools

import jax
import jax.numpy as jnp
from jax import lax
from jax.experimental import pallas as pl
from jax.experimental.pallas import tpu as pltpu
from jax.experimental.pallas import tpu_sc as plsc

VOCAB = 10000
EMB_DIM = 64
BATCH = 16384
SEQ = 20
OUT_DIM = EMB_DIM + 1
YEAR_MEAN = 1990.0
INV_STD = float(1.0 / (400.0 + 1e-7) ** 0.5)

NC = 2
NS = 16
L = 16
NW = NC * NS
ROWS_W = BATCH // NW
CHUNK = 64
STEPS = ROWS_W // CHUNK
IDX_W = CHUNK * SEQ
IDX_MINOR = 128
IDX_ROWS = IDX_W // IDX_MINOR
GROUPS = CHUNK // L
NH = EMB_DIM // 32

_mesh = plsc.VectorSubcoreMesh(
    core_axis_name="c", subcore_axis_name="s", num_cores=NC, num_subcores=NS
)


@functools.partial(
    pl.kernel,
    out_type=jax.ShapeDtypeStruct((BATCH * OUT_DIM,), jnp.float32),
    mesh=_mesh,
    scratch_types=[
        pltpu.VMEM((ROWS_W * SEQ,), jnp.int32),
        pltpu.VMEM((IDX_W, EMB_DIM), jnp.bfloat16),
        pltpu.VMEM((IDX_W, EMB_DIM), jnp.bfloat16),
        pltpu.VMEM((ROWS_W,), jnp.float32),
        pltpu.VMEM((ROWS_W,), jnp.float32),
        pltpu.VMEM((ROWS_W,), jnp.float32),
        pltpu.VMEM((CHUNK * OUT_DIM,), jnp.float32),
        pltpu.VMEM((8, EMB_DIM), jnp.bfloat16),
        pltpu.VMEM_SHARED((VOCAB, EMB_DIM), jnp.bfloat16),
        pltpu.SemaphoreType.DMA,
        pltpu.SemaphoreType.DMA,
    ],
    compiler_params=pltpu.CompilerParams(
        needs_layout_passes=False, use_tc_tiling_on_sc=False),
)
def _sc_pool(title_hbm, year_hbm, table_hbm, out_hbm,
             tok_v, rows0_v, rows1_v, year_v, nz_v, rc_v, out_v, t0_v,
             table_sh, sem0, sem1):
    wid = lax.axis_index("s") * NC + lax.axis_index("c")
    iota = lax.iota(jnp.int32, L)
    base = wid * ROWS_W
    rows_v = (rows0_v, rows1_v)
    sems = (sem0, sem1)

    @pl.when(lax.axis_index("s") == 0)
    def _():
        pltpu.sync_copy(table_hbm, table_sh)

    pltpu.sync_copy(title_hbm.at[pl.ds(base * SEQ, ROWS_W * SEQ)], tok_v)
    pltpu.sync_copy(year_hbm.at[pl.ds(base, ROWS_W)], year_v)
    pltpu.sync_copy(table_hbm.at[pl.ds(0, 8)], t0_v)
    t0 = [plsc.unpack(t0_v[0, pl.ds(h * 32, 32)],
                      format=plsc.PackFormat.INTERLEAVED) for h in range(NH)]

    def issue_gathers(t, p):
        pltpu.async_copy(
            table_sh.at[tok_v.at[pl.ds(t * IDX_W, IDX_W)]],
            rows_v[p], sems[p])

    plsc.subcore_barrier()
    issue_gathers(0, 0)

    for g in range(ROWS_W // L):
        r_idx = g * L + iota
        cnt = jnp.zeros((L,), jnp.float32)
        for t in range(SEQ):
            tv = plsc.load_gather(tok_v, [r_idx * SEQ + t])
            cnt = cnt + jnp.where(tv == 0, 1.0, 0.0)
        nz_v[pl.ds(g * L, L)] = cnt
        rc_v[pl.ds(g * L, L)] = 1.0 / jnp.maximum(float(SEQ) - cnt, 1.0)

    def process(s, p):
        nxt = s + 1

        @pl.when(nxt < STEPS)
        def _():
            issue_gathers(nxt, 1 - p)

        for g in range(GROUPS):
            r_idx = g * L + iota
            yn = (year_v[pl.ds(s * CHUNK + g * L, L)] - YEAR_MEAN) * INV_STD
            plsc.store_scatter(out_v, [r_idx * OUT_DIM + EMB_DIM], yn)

        pltpu.make_async_copy(table_sh.at[pl.ds(0, IDX_W)],
                              rows_v[p], sems[p]).wait()

        def row(r, carry2):
            gr = s * CHUNK + r
            nzr = plsc.load_gather(nz_v, [jnp.full((L,), gr, jnp.int32)])
            rcr = plsc.load_gather(rc_v, [jnp.full((L,), gr, jnp.int32)])
            accp = [jnp.zeros((2 * L,), jnp.bfloat16) for _ in range(NH)]
            for t in range(SEQ):
                src = r * SEQ + t
                for h in range(NH):
                    accp[h] = accp[h] + rows_v[p][src, pl.ds(h * 32, 32)]
            for h in range(NH):
                a, b = plsc.unpack(accp[h], format=plsc.PackFormat.INTERLEAVED)
                for e, acc in enumerate((a, b)):
                    val = (acc - nzr * t0[h][e]) * rcr
                    plsc.store_scatter(
                        out_v, [r * OUT_DIM + h * 32 + 2 * iota + e], val)
            return carry2

        lax.fori_loop(0, CHUNK, row, 0)
        pltpu.sync_copy(
            out_v,
            out_hbm.at[pl.ds((base + s * CHUNK) * OUT_DIM, CHUNK * OUT_DIM)])

    def pair(s2, carry):
        process(2 * s2, 0)
        process(2 * s2 + 1, 1)
        return carry

    lax.fori_loop(0, STEPS // 2, pair, 0)


def kernel(movie_title, movie_Year, table):
    out_flat = _sc_pool(movie_title.reshape(-1), movie_Year,
                        table.astype(jnp.bfloat16))
    return out_flat.reshape(BATCH, OUT_DIM)


# speedup vs baseline: 1.1600x; 1.1600x over previous
import functools

import jax
import jax.numpy as jnp
from jax import lax
from jax.experimental import pallas as pl
from jax.experimental.pallas import tpu as pltpu
from jax.experimental.pallas import tpu_sc as plsc

VOCAB = 10000
EMB_DIM = 64
BATCH = 16384
SEQ = 20
OUT_DIM = EMB_DIM + 1
OUT_PAD = 128
YEAR_MEAN = 1990.0
INV_STD = float(1.0 / (400.0 + 1e-7) ** 0.5)

NC = 2
NS = 16
L = 16
NW = NC * NS
ROWS_W = BATCH // NW
CHUNK = 64
STEPS = ROWS_W // CHUNK
IDX_W = CHUNK * SEQ
IDX_MINOR = 128
IDX_ROWS = IDX_W // IDX_MINOR
GROUPS = CHUNK // L
NH = EMB_DIM // 32

_mesh = plsc.VectorSubcoreMesh(
    core_axis_name="c", subcore_axis_name="s", num_cores=NC, num_subcores=NS
)


@functools.partial(
    pl.kernel,
    out_type=jax.ShapeDtypeStruct((BATCH * OUT_PAD,), jnp.float32),
    mesh=_mesh,
    scratch_types=[
        pltpu.VMEM((ROWS_W * SEQ,), jnp.int32),
        pltpu.VMEM((IDX_W, EMB_DIM), jnp.bfloat16),
        pltpu.VMEM((IDX_W, EMB_DIM), jnp.bfloat16),
        pltpu.VMEM((ROWS_W,), jnp.float32),
        pltpu.VMEM((ROWS_W,), jnp.float32),
        pltpu.VMEM((ROWS_W,), jnp.float32),
        pltpu.VMEM((CHUNK * OUT_PAD,), jnp.float32),
        pltpu.VMEM((8, EMB_DIM), jnp.bfloat16),
        pltpu.VMEM_SHARED((VOCAB, EMB_DIM), jnp.bfloat16),
        pltpu.SemaphoreType.DMA,
        pltpu.SemaphoreType.DMA,
    ],
    compiler_params=pltpu.CompilerParams(
        needs_layout_passes=False, use_tc_tiling_on_sc=False),
)
def _sc_pool(title_hbm, year_hbm, table_hbm, out_hbm,
             tok_v, rows0_v, rows1_v, year_v, nz_v, rc_v, out_v, t0_v,
             table_sh, sem0, sem1):
    wid = lax.axis_index("s") * NC + lax.axis_index("c")
    iota = lax.iota(jnp.int32, L)
    base = wid * ROWS_W
    rows_v = (rows0_v, rows1_v)
    sems = (sem0, sem1)

    @pl.when(lax.axis_index("s") == 0)
    def _():
        pltpu.sync_copy(table_hbm, table_sh)

    pltpu.sync_copy(title_hbm.at[pl.ds(base * SEQ, ROWS_W * SEQ)], tok_v)
    pltpu.sync_copy(year_hbm.at[pl.ds(base, ROWS_W)], year_v)
    pltpu.sync_copy(table_hbm.at[pl.ds(0, 8)], t0_v)
    t0 = [plsc.unpack(t0_v[0, pl.ds(h * 32, 32)],
                      format=plsc.PackFormat.INTERLEAVED) for h in range(NH)]

    def issue_gathers(t, p):
        pltpu.async_copy(
            table_sh.at[tok_v.at[pl.ds(t * IDX_W, IDX_W)]],
            rows_v[p], sems[p])

    plsc.subcore_barrier()
    issue_gathers(0, 0)

    for g in range(ROWS_W // L):
        r_idx = g * L + iota
        cnt = jnp.zeros((L,), jnp.float32)
        for t in range(SEQ):
            tv = plsc.load_gather(tok_v, [r_idx * SEQ + t])
            cnt = cnt + jnp.where(tv == 0, 1.0, 0.0)
        nz_v[pl.ds(g * L, L)] = cnt
        rc_v[pl.ds(g * L, L)] = 1.0 / jnp.maximum(float(SEQ) - cnt, 1.0)

    def process(s, p):
        nxt = s + 1

        @pl.when(nxt < STEPS)
        def _():
            issue_gathers(nxt, 1 - p)

        for g in range(GROUPS):
            r_idx = g * L + iota
            yn = (year_v[pl.ds(s * CHUNK + g * L, L)] - YEAR_MEAN) * INV_STD
            plsc.store_scatter(out_v, [r_idx * OUT_PAD + EMB_DIM], yn)

        pltpu.make_async_copy(table_sh.at[pl.ds(0, IDX_W)],
                              rows_v[p], sems[p]).wait()

        def row(r, carry2):
            gr = s * CHUNK + r
            nzr = plsc.load_gather(nz_v, [jnp.full((L,), gr, jnp.int32)])
            rcr = plsc.load_gather(rc_v, [jnp.full((L,), gr, jnp.int32)])
            accp = [jnp.zeros((2 * L,), jnp.bfloat16) for _ in range(NH)]
            for t in range(SEQ):
                src = r * SEQ + t
                for h in range(NH):
                    accp[h] = accp[h] + rows_v[p][src, pl.ds(h * 32, 32)]
            for h in range(NH):
                a, b = plsc.unpack(accp[h], format=plsc.PackFormat.INTERLEAVED)
                for e, acc in enumerate((a, b)):
                    val = (acc - nzr * t0[h][e]) * rcr
                    plsc.store_scatter(
                        out_v, [r * OUT_PAD + h * 32 + 2 * iota + e], val)
            return carry2

        lax.fori_loop(0, CHUNK, row, 0)
        pltpu.sync_copy(
            out_v,
            out_hbm.at[pl.ds((base + s * CHUNK) * OUT_PAD, CHUNK * OUT_PAD)])

    def pair(s2, carry):
        process(2 * s2, 0)
        process(2 * s2 + 1, 1)
        return carry

    lax.fori_loop(0, STEPS // 2, pair, 0)


def kernel(movie_title, movie_Year, table):
    out_flat = _sc_pool(movie_title.reshape(-1), movie_Year,
                        table.astype(jnp.bfloat16))
    return out_flat.reshape(BATCH, OUT_PAD)[:, :OUT_DIM]


# speedup vs baseline: 1.1920x; 1.0277x over previous
import functools

import jax
import jax.numpy as jnp
from jax import lax
from jax.experimental import pallas as pl
from jax.experimental.pallas import tpu as pltpu
from jax.experimental.pallas import tpu_sc as plsc

VOCAB = 10000
EMB_DIM = 64
BATCH = 16384
SEQ = 20
OUT_DIM = EMB_DIM + 1
OUT_PAD = 128
YEAR_MEAN = 1990.0
INV_STD = float(1.0 / (400.0 + 1e-7) ** 0.5)

NC = 2
NS = 16
L = 16
NW = NC * NS
ROWS_W = BATCH // NW
CHUNK = 64
STEPS = ROWS_W // CHUNK
IDX_W = CHUNK * SEQ
IDX_MINOR = 128
IDX_ROWS = IDX_W // IDX_MINOR
GROUPS = CHUNK // L
NH = EMB_DIM // 32

_mesh = plsc.VectorSubcoreMesh(
    core_axis_name="c", subcore_axis_name="s", num_cores=NC, num_subcores=NS
)


@functools.partial(
    pl.kernel,
    out_type=jax.ShapeDtypeStruct((BATCH * OUT_PAD,), jnp.float32),
    mesh=_mesh,
    scratch_types=[
        pltpu.VMEM((ROWS_W * SEQ,), jnp.int32),
        pltpu.VMEM((IDX_W, EMB_DIM), jnp.bfloat16),
        pltpu.VMEM((IDX_W, EMB_DIM), jnp.bfloat16),
        pltpu.VMEM((ROWS_W,), jnp.float32),
        pltpu.VMEM((ROWS_W,), jnp.float32),
        pltpu.VMEM((ROWS_W,), jnp.float32),
        pltpu.VMEM((CHUNK * OUT_PAD,), jnp.float32),
        pltpu.VMEM((CHUNK * OUT_PAD,), jnp.float32),
        pltpu.VMEM((8, EMB_DIM), jnp.bfloat16),
        pltpu.VMEM_SHARED((VOCAB, EMB_DIM), jnp.bfloat16),
        pltpu.SemaphoreType.DMA,
        pltpu.SemaphoreType.DMA,
        pltpu.SemaphoreType.DMA,
        pltpu.SemaphoreType.DMA,
    ],
    compiler_params=pltpu.CompilerParams(
        needs_layout_passes=False, use_tc_tiling_on_sc=False),
)
def _sc_pool(title_hbm, year_hbm, table_hbm, out_hbm,
             tok_v, rows0_v, rows1_v, year_v, nz_v, rc_v, out0_v, out1_v,
             t0_v, table_sh, sem0, sem1, semo0, semo1):
    sid = lax.axis_index("s")
    wid = sid * NC + lax.axis_index("c")
    iota = lax.iota(jnp.int32, L)
    base = wid * ROWS_W
    rows_v = (rows0_v, rows1_v)
    out_vs = (out0_v, out1_v)
    sems = (sem0, sem1)
    semos = (semo0, semo1)

    trows = VOCAB // NS
    pltpu.sync_copy(table_hbm.at[pl.ds(sid * trows, trows)],
                    table_sh.at[pl.ds(sid * trows, trows)])

    pltpu.sync_copy(title_hbm.at[pl.ds(base * SEQ, ROWS_W * SEQ)], tok_v)
    pltpu.sync_copy(year_hbm.at[pl.ds(base, ROWS_W)], year_v)
    pltpu.sync_copy(table_hbm.at[pl.ds(0, 8)], t0_v)
    t0 = [plsc.unpack(t0_v[0, pl.ds(h * 32, 32)],
                      format=plsc.PackFormat.INTERLEAVED) for h in range(NH)]

    def issue_gathers(t, p):
        pltpu.async_copy(
            table_sh.at[tok_v.at[pl.ds(t * IDX_W, IDX_W)]],
            rows_v[p], sems[p])

    plsc.subcore_barrier()
    issue_gathers(0, 0)

    for g in range(ROWS_W // L):
        r_idx = g * L + iota
        cnt = jnp.zeros((L,), jnp.float32)
        for t in range(SEQ):
            tv = plsc.load_gather(tok_v, [r_idx * SEQ + t])
            cnt = cnt + jnp.where(tv == 0, 1.0, 0.0)
        nz_v[pl.ds(g * L, L)] = cnt
        rc_v[pl.ds(g * L, L)] = 1.0 / jnp.maximum(float(SEQ) - cnt, 1.0)

    def process(s, p):
        out_v = out_vs[p]
        nxt = s + 1

        @pl.when(nxt < STEPS)
        def _():
            issue_gathers(nxt, 1 - p)

        @pl.when(s >= 2)
        def _():
            pltpu.make_async_copy(
                out_v,
                out_hbm.at[pl.ds((base + (s - 2) * CHUNK) * OUT_PAD,
                                 CHUNK * OUT_PAD)],
                semos[p]).wait()

        for g in range(GROUPS):
            r_idx = g * L + iota
            yn = (year_v[pl.ds(s * CHUNK + g * L, L)] - YEAR_MEAN) * INV_STD
            plsc.store_scatter(out_v, [r_idx * OUT_PAD + EMB_DIM], yn)

        pltpu.make_async_copy(table_sh.at[pl.ds(0, IDX_W)],
                              rows_v[p], sems[p]).wait()

        def row(r, carry2):
            gr = s * CHUNK + r
            nzr = plsc.load_gather(nz_v, [jnp.full((L,), gr, jnp.int32)])
            rcr = plsc.load_gather(rc_v, [jnp.full((L,), gr, jnp.int32)])
            accp = [rows_v[p][r * SEQ, pl.ds(h * 32, 32)] for h in range(NH)]
            for t in range(1, SEQ):
                src = r * SEQ + t
                for h in range(NH):
                    accp[h] = accp[h] + rows_v[p][src, pl.ds(h * 32, 32)]
            for h in range(NH):
                a, b = plsc.unpack(accp[h], format=plsc.PackFormat.INTERLEAVED)
                for e, acc in enumerate((a, b)):
                    val = (acc - nzr * t0[h][e]) * rcr
                    plsc.store_scatter(
                        out_v, [r * OUT_PAD + h * 32 + 2 * iota + e], val)
            return carry2

        lax.fori_loop(0, CHUNK, row, 0)
        pltpu.async_copy(
            out_v,
            out_hbm.at[pl.ds((base + s * CHUNK) * OUT_PAD, CHUNK * OUT_PAD)],
            semos[p])

    def pair(s2, carry):
        process(2 * s2, 0)
        process(2 * s2 + 1, 1)
        return carry

    lax.fori_loop(0, STEPS // 2, pair, 0)

    for p in range(2):
        pltpu.make_async_copy(
            out_vs[p],
            out_hbm.at[pl.ds((base + (STEPS - 2 + p) * CHUNK) * OUT_PAD,
                             CHUNK * OUT_PAD)],
            semos[p]).wait()


def kernel(movie_title, movie_Year, table):
    out_flat = _sc_pool(movie_title.reshape(-1), movie_Year,
                        table.astype(jnp.bfloat16))
    return out_flat.reshape(BATCH, OUT_PAD)[:, :OUT_DIM]
